# baseline (speedup 1.0000x reference)
import functools

import jax
import jax.numpy as jnp
from jax import lax
from jax.experimental import pallas as pl
from jax.experimental.pallas import tpu as pltpu
from jax.experimental.pallas import tpu_sc as plsc

INPUT = 64
HID = 128
BATCH = 4096
SEQ = 200
N_TOK = BATCH * SEQ
CHUNK = 128
N_CHUNKS = N_TOK // CHUNK
NW = 32
CHUNKS_PER_W = N_CHUNKS // NW
RING = 4
N_GROUPS = CHUNKS_PER_W // RING
REP = 1024

BB = 64
PAIRS = N_TOK // 2
BPAIR = BB * SEQ // 2

TBLK = 512
HALF = 977 * TBLK


def _tc_transpose(table_t):
  def body(lo_ref, hi_ref, o_ref):
    lo = lo_ref[...]
    hi = hi_ref[...]
    o_ref[...] = jnp.concatenate([lo.T, hi.T], axis=1)

  return pl.pallas_call(
      body,
      grid=(HALF // TBLK,),
      in_specs=[
          pl.BlockSpec((INPUT, TBLK), lambda j: (0, j)),
          pl.BlockSpec((INPUT, TBLK), lambda j: (0, j + HALF // TBLK)),
      ],
      out_specs=pl.BlockSpec((TBLK, 2 * INPUT), lambda j: (j, 0)),
      out_shape=jax.ShapeDtypeStruct((HALF, 2 * INPUT), jnp.float32),
  )(table_t, table_t)


def _sc_gather(table_id, table_rt, idx2d, idxr2d):
  mesh = plsc.VectorSubcoreMesh(core_axis_name="c", subcore_axis_name="s")

  @functools.partial(
      pl.kernel,
      out_type=(
          jax.ShapeDtypeStruct((N_CHUNKS, CHUNK, INPUT), jnp.float32),
          jax.ShapeDtypeStruct((N_CHUNKS, CHUNK, INPUT), jnp.float32),
      ),
      mesh=mesh,
      scratch_types=[
          pltpu.VMEM((CHUNKS_PER_W, CHUNK), jnp.int32),
          pltpu.VMEM((CHUNKS_PER_W, CHUNK), jnp.int32),
          pltpu.VMEM((RING, CHUNK, INPUT), jnp.float32),
          pltpu.VMEM((RING, CHUNK, INPUT), jnp.float32),
          pltpu.SemaphoreType.DMA,
          pltpu.SemaphoreType.DMA,
      ],
      compiler_params=pltpu.CompilerParams(use_tc_tiling_on_sc=False),
  )
  def k(tid_hbm, trt_hbm, idx_hbm, idxr_hbm, oid_hbm, ort_hbm,
        idx_v, idxr_v, bid_v, brt_v, gsem, wsem):
    wid = lax.axis_index("s") * 2 + lax.axis_index("c")
    base = wid * CHUNKS_PER_W
    pltpu.sync_copy(idx_hbm.at[pl.ds(base, CHUNKS_PER_W)], idx_v)
    pltpu.sync_copy(idxr_hbm.at[pl.ds(base, CHUNKS_PER_W)], idxr_v)

    def group(g, _):
      @pl.when(g > 0)
      def _():
        for r in range(RING):
          pltpu.make_async_copy(bid_v.at[r], oid_hbm.at[base], wsem).wait()
          pltpu.make_async_copy(brt_v.at[r], ort_hbm.at[base], wsem).wait()
      handles = []
      for r in range(RING):
        c = g * RING + r
        handles.append(pltpu.make_async_copy(
            tid_hbm.at[idx_v.at[c]], bid_v.at[r], gsem))
        handles.append(pltpu.make_async_copy(
            trt_hbm.at[idxr_v.at[c]], brt_v.at[r], gsem))
      for h in handles:
        h.start()
      for h in handles:
        h.wait()
      for r in range(RING):
        c = g * RING + r
        pltpu.make_async_copy(bid_v.at[r], oid_hbm.at[base + c], wsem).start()
        pltpu.make_async_copy(brt_v.at[r], ort_hbm.at[base + c], wsem).start()
      return 0

    lax.fori_loop(0, N_GROUPS, group, 0)
    for r in range(RING):
      pltpu.make_async_copy(bid_v.at[r], oid_hbm.at[base], wsem).wait()
      pltpu.make_async_copy(brt_v.at[r], ort_hbm.at[base], wsem).wait()

  return k(table_id, table_rt, idx2d, idxr2d)


def _tc_proj(gid2, grt2, seg2, b2, wid2b, wrt2b, interpret=False):
  def body(gid_ref, grt_ref, seg_ref, b_ref, wid_ref, wrt_ref, o_ref):
    h = jnp.dot(gid_ref[...], wid_ref[...],
                preferred_element_type=jnp.float32)
    h = h + jnp.dot(grt_ref[...], wrt_ref[...],
                    preferred_element_type=jnp.float32)
    h = h + b_ref[0:1, :]
    h = jnp.maximum(h, 0.0)
    hsum = h[:, :HID] + h[:, HID:]
    o_ref[...] = jnp.dot(seg_ref[...], hsum,
                         preferred_element_type=jnp.float32)

  return pl.pallas_call(
      body,
      grid=(BATCH // BB,),
      in_specs=[
          pl.BlockSpec((BPAIR, 2 * INPUT), lambda i: (i, 0)),
          pl.BlockSpec((BPAIR, 2 * INPUT), lambda i: (i, 0)),
          pl.BlockSpec((BB, BPAIR), lambda i: (0, 0)),
          pl.BlockSpec((8, 2 * HID), lambda i: (0, 0)),
          pl.BlockSpec((2 * INPUT, 2 * HID), lambda i: (0, 0)),
          pl.BlockSpec((2 * INPUT, 2 * HID), lambda i: (0, 0)),
      ],
      out_specs=pl.BlockSpec((BB, HID), lambda i: (i, 0)),
      out_shape=jax.ShapeDtypeStruct((BATCH, HID), jnp.float32),
      interpret=interpret,
  )(gid2, grt2, seg2, b2, wid2b, wrt2b)


def kernel(x, ids, ratings_emd, id_emd, W, b):
  wrt_t = W[:, :INPUT].T
  wid_t = W[:, INPUT:].T
  zz = jnp.zeros((INPUT, HID), jnp.float32)
  wid2b = jnp.block([[wid_t, zz], [zz, wid_t]])
  wrt2b = jnp.block([[wrt_t, zz], [zz, wrt_t]])
  b2 = jnp.broadcast_to(jnp.concatenate([b, b])[None, :], (8, 2 * HID))
  seg2 = (lax.broadcasted_iota(jnp.int32, (BB, BPAIR), 1) // (SEQ // 2) ==
          lax.broadcasted_iota(jnp.int32, (BB, BPAIR), 0)
          ).astype(jnp.float32) * (1.0 / SEQ)
  table_rt = jnp.repeat(ratings_emd, REP, axis=0)
  table_lin = _tc_transpose(id_emd.T).reshape(2 * HALF, INPUT)
  ids_m = jnp.where(ids < HALF, 2 * ids, 2 * (ids - HALF) + 1)
  idx2d = ids_m.reshape(N_CHUNKS, CHUNK)
  x2d = x.reshape(N_CHUNKS, CHUNK)
  pos2d = (lax.broadcasted_iota(jnp.int32, (N_CHUNKS, CHUNK), 0) % 8) * CHUNK \
      + lax.broadcasted_iota(jnp.int32, (N_CHUNKS, CHUNK), 1)
  idxr2d = x2d * REP + pos2d
  gid, grt = _sc_gather(table_lin, table_rt, idx2d, idxr2d)
  gid2 = gid.reshape(PAIRS, 2 * INPUT)
  grt2 = grt.reshape(PAIRS, 2 * INPUT)
  return _tc_proj(gid2, grt2, seg2, b2, wid2b, wrt2b)


# speedup vs baseline: 1.0979x; 1.0979x over previous
import functools

import jax
import jax.numpy as jnp
from jax import lax
from jax.experimental import pallas as pl
from jax.experimental.pallas import tpu as pltpu
from jax.experimental.pallas import tpu_sc as plsc

INPUT = 64
HID = 128
BATCH = 4096
SEQ = 200
N_TOK = BATCH * SEQ
CHUNK = 128
N_CHUNKS = N_TOK // CHUNK
NW = 32
CHUNKS_PER_W = N_CHUNKS // NW
RING = 4
N_GROUPS = CHUNKS_PER_W // RING

BB = 64
PAIRS = N_TOK // 2
BPAIR = BB * SEQ // 2

TBLK = 512
HALF = 977 * TBLK


def _tc_transpose(table_t):
  def body(lo_ref, hi_ref, o_ref):
    lo = lo_ref[...]
    hi = hi_ref[...]
    o_ref[...] = jnp.concatenate([lo.T, hi.T], axis=1)

  return pl.pallas_call(
      body,
      grid=(HALF // TBLK,),
      in_specs=[
          pl.BlockSpec((INPUT, TBLK), lambda j: (0, j)),
          pl.BlockSpec((INPUT, TBLK), lambda j: (0, j + HALF // TBLK)),
      ],
      out_specs=pl.BlockSpec((TBLK, 2 * INPUT), lambda j: (j, 0)),
      out_shape=jax.ShapeDtypeStruct((HALF, 2 * INPUT), jnp.float32),
  )(table_t, table_t)


def _sc_gather(table_id, idx2d):
  mesh = plsc.VectorSubcoreMesh(core_axis_name="c", subcore_axis_name="s")

  @functools.partial(
      pl.kernel,
      out_type=jax.ShapeDtypeStruct((N_CHUNKS, CHUNK, INPUT), jnp.float32),
      mesh=mesh,
      scratch_types=[
          pltpu.VMEM((CHUNKS_PER_W, CHUNK), jnp.int32),
          pltpu.VMEM((RING, CHUNK, INPUT), jnp.float32),
          pltpu.SemaphoreType.DMA,
          pltpu.SemaphoreType.DMA,
      ],
      compiler_params=pltpu.CompilerParams(use_tc_tiling_on_sc=False),
  )
  def k(tid_hbm, idx_hbm, oid_hbm, idx_v, bid_v, gsem, wsem):
    wid = lax.axis_index("s") * 2 + lax.axis_index("c")
    base = wid * CHUNKS_PER_W
    pltpu.sync_copy(idx_hbm.at[pl.ds(base, CHUNKS_PER_W)], idx_v)

    def group(g, _):
      @pl.when(g > 0)
      def _():
        for r in range(RING):
          pltpu.make_async_copy(bid_v.at[r], oid_hbm.at[base], wsem).wait()
      handles = []
      for r in range(RING):
        c = g * RING + r
        handles.append(pltpu.make_async_copy(
            tid_hbm.at[idx_v.at[c]], bid_v.at[r], gsem))
      for h in handles:
        h.start()
      for h in handles:
        h.wait()
      for r in range(RING):
        c = g * RING + r
        pltpu.make_async_copy(bid_v.at[r], oid_hbm.at[base + c], wsem).start()
      return 0

    lax.fori_loop(0, N_GROUPS, group, 0)
    for r in range(RING):
      pltpu.make_async_copy(bid_v.at[r], oid_hbm.at[base], wsem).wait()

  return k(table_id, idx2d)


def _tc_proj(gid2, x2, seg2, rp2, wid2b, interpret=False):
  def body(gid_ref, x_ref, seg_ref, rp_ref, wid_ref, o_ref):
    h = jnp.dot(gid_ref[...], wid_ref[...],
                preferred_element_type=jnp.float32)
    j16 = lax.broadcasted_iota(jnp.int32, (BPAIR, 16), 1)
    tgt = jnp.where(j16 < 8, x_ref[:, 0:1], x_ref[:, 1:2] + 8)
    oh = (j16 == tgt).astype(jnp.float32)
    h = h + jnp.dot(oh, rp_ref[...],
                    preferred_element_type=jnp.float32)
    h = jnp.maximum(h, 0.0)
    hsum = h[:, :HID] + h[:, HID:]
    o_ref[...] = jnp.dot(seg_ref[...], hsum,
                         preferred_element_type=jnp.float32)

  return pl.pallas_call(
      body,
      grid=(BATCH // BB,),
      in_specs=[
          pl.BlockSpec((BPAIR, 2 * INPUT), lambda i: (i, 0)),
          pl.BlockSpec((BPAIR, 2), lambda i: (i, 0)),
          pl.BlockSpec((BB, BPAIR), lambda i: (0, 0)),
          pl.BlockSpec((16, 2 * HID), lambda i: (0, 0)),
          pl.BlockSpec((2 * INPUT, 2 * HID), lambda i: (0, 0)),
      ],
      out_specs=pl.BlockSpec((BB, HID), lambda i: (i, 0)),
      out_shape=jax.ShapeDtypeStruct((BATCH, HID), jnp.float32),
      interpret=interpret,
  )(gid2, x2, seg2, rp2, wid2b)


def kernel(x, ids, ratings_emd, id_emd, W, b):
  wrt_t = W[:, :INPUT].T
  wid_t = W[:, INPUT:].T
  zz = jnp.zeros((INPUT, HID), jnp.float32)
  wid2b = jnp.block([[wid_t, zz], [zz, wid_t]])
  rp = ratings_emd @ wrt_t + b[None, :]
  rp2 = jnp.zeros((16, 2 * HID), jnp.float32)
  rp2 = rp2.at[0:6, :HID].set(rp).at[8:14, HID:].set(rp)
  seg2 = (lax.broadcasted_iota(jnp.int32, (BB, BPAIR), 1) // (SEQ // 2) ==
          lax.broadcasted_iota(jnp.int32, (BB, BPAIR), 0)
          ).astype(jnp.float32) * (1.0 / SEQ)
  table_lin = _tc_transpose(id_emd.T).reshape(2 * HALF, INPUT)
  ids_m = jnp.where(ids < HALF, 2 * ids, 2 * (ids - HALF) + 1)
  idx2d = ids_m.reshape(N_CHUNKS, CHUNK)
  gid = _sc_gather(table_lin, idx2d)
  gid2 = gid.reshape(PAIRS, 2 * INPUT)
  x2 = x.reshape(PAIRS, 2)
  return _tc_proj(gid2, x2, seg2, rp2, wid2b)


# speedup vs baseline: 1.2055x; 1.0980x over previous
import functools

import jax
import jax.numpy as jnp
from jax import lax
from jax.experimental import pallas as pl
from jax.experimental.pallas import tpu as pltpu
from jax.experimental.pallas import tpu_sc as plsc

INPUT = 64
HID = 128
BATCH = 4096
SEQ = 200
N_TOK = BATCH * SEQ
CHUNK = 128
N_CHUNKS = N_TOK // CHUNK
NW = 32
CHUNKS_PER_W = N_CHUNKS // NW
RING = 4
N_GROUPS = CHUNKS_PER_W // RING

BB = 64
PAIRS = N_TOK // 2
BPAIR = BB * SEQ // 2

def _sc_gather(table_id, idx2d):
  mesh = plsc.VectorSubcoreMesh(core_axis_name="c", subcore_axis_name="s")

  @functools.partial(
      pl.kernel,
      out_type=jax.ShapeDtypeStruct((N_CHUNKS, CHUNK, INPUT), jnp.float32),
      mesh=mesh,
      scratch_types=[
          pltpu.VMEM((CHUNKS_PER_W, CHUNK), jnp.int32),
          pltpu.VMEM((RING, CHUNK, INPUT), jnp.float32),
          pltpu.SemaphoreType.DMA,
          pltpu.SemaphoreType.DMA,
      ],
      compiler_params=pltpu.CompilerParams(use_tc_tiling_on_sc=False),
  )
  def k(tid_hbm, idx_hbm, oid_hbm, idx_v, bid_v, gsem, wsem):
    wid = lax.axis_index("s") * 2 + lax.axis_index("c")
    base = wid * CHUNKS_PER_W
    pltpu.sync_copy(idx_hbm.at[pl.ds(base, CHUNKS_PER_W)], idx_v)

    def group(g, _):
      @pl.when(g > 0)
      def _():
        for r in range(RING):
          pltpu.make_async_copy(bid_v.at[r], oid_hbm.at[base], wsem).wait()
      handles = []
      for r in range(RING):
        c = g * RING + r
        handles.append(pltpu.make_async_copy(
            tid_hbm.at[idx_v.at[c]], bid_v.at[r], gsem))
      for h in handles:
        h.start()
      for h in handles:
        h.wait()
      for r in range(RING):
        c = g * RING + r
        pltpu.make_async_copy(bid_v.at[r], oid_hbm.at[base + c], wsem).start()
      return 0

    lax.fori_loop(0, N_GROUPS, group, 0)
    for r in range(RING):
      pltpu.make_async_copy(bid_v.at[r], oid_hbm.at[base], wsem).wait()

  return k(table_id, idx2d)


def _tc_proj(gid2, x2, seg2, rp2, wid2b, interpret=False):
  def body(gid_ref, x_ref, seg_ref, rp_ref, wid_ref, o_ref):
    h = jnp.dot(gid_ref[...], wid_ref[...],
                preferred_element_type=jnp.float32)
    j16 = lax.broadcasted_iota(jnp.int32, (BPAIR, 16), 1)
    tgt = jnp.where(j16 < 8, x_ref[:, 0:1], x_ref[:, 1:2] + 8)
    oh = (j16 == tgt).astype(jnp.float32)
    h = h + jnp.dot(oh, rp_ref[...],
                    preferred_element_type=jnp.float32)
    h = jnp.maximum(h, 0.0)
    hsum = h[:, :HID] + h[:, HID:]
    o_ref[...] = jnp.dot(seg_ref[...], hsum,
                         preferred_element_type=jnp.float32)

  return pl.pallas_call(
      body,
      grid=(BATCH // BB,),
      in_specs=[
          pl.BlockSpec((BPAIR, 2 * INPUT), lambda i: (i, 0)),
          pl.BlockSpec((BPAIR, 2), lambda i: (i, 0)),
          pl.BlockSpec((BB, BPAIR), lambda i: (0, 0)),
          pl.BlockSpec((16, 2 * HID), lambda i: (0, 0)),
          pl.BlockSpec((2 * INPUT, 2 * HID), lambda i: (0, 0)),
      ],
      out_specs=pl.BlockSpec((BB, HID), lambda i: (i, 0)),
      out_shape=jax.ShapeDtypeStruct((BATCH, HID), jnp.float32),
      interpret=interpret,
  )(gid2, x2, seg2, rp2, wid2b)


def kernel(x, ids, ratings_emd, id_emd, W, b):
  wrt_t = W[:, :INPUT].T
  wid_t = W[:, INPUT:].T
  zz = jnp.zeros((INPUT, HID), jnp.float32)
  wid2b = jnp.block([[wid_t, zz], [zz, wid_t]])
  rp = ratings_emd @ wrt_t + b[None, :]
  rp2 = jnp.zeros((16, 2 * HID), jnp.float32)
  rp2 = rp2.at[0:6, :HID].set(rp).at[8:14, HID:].set(rp)
  seg2 = (lax.broadcasted_iota(jnp.int32, (BB, BPAIR), 1) // (SEQ // 2) ==
          lax.broadcasted_iota(jnp.int32, (BB, BPAIR), 0)
          ).astype(jnp.float32) * (1.0 / SEQ)
  idx2d = ids.reshape(N_CHUNKS, CHUNK)
  gid = _sc_gather(id_emd, idx2d)
  gid2 = gid.reshape(PAIRS, 2 * INPUT)
  x2 = x.reshape(PAIRS, 2)
  return _tc_proj(gid2, x2, seg2, rp2, wid2b)
